# Initial kernel scaffold; baseline (speedup 1.0000x reference)
import jax
import jax.numpy as jnp
from jax.experimental import pallas as pl


def kernel(x, table, W, b):
    raise NotImplementedError("write your pallas kernel here")


# speedup vs baseline: 3.5505x; 3.5505x over previous
import functools

import numpy as np
import jax
import jax.numpy as jnp
from jax import lax
from jax.experimental import pallas as pl
from jax.experimental.pallas import tpu as pltpu
from jax.experimental.pallas import tpu_sc as plsc

_NUM_CORES = 2
_NUM_SUBCORES = 16
_NUM_WORKERS = _NUM_CORES * _NUM_SUBCORES
_LANES = 16

_CHUNK = 2048
_D = 10
_BLOCK_ROWS = 8
_PHASES = 5


def _fuse_table_body(table_ref, w_ref, b_ref, out_ref):
    out_ref[...] = (
        lax.dot_general(
            table_ref[...], w_ref[...],
            dimension_numbers=(((1,), (1,)), ((), ())),
            preferred_element_type=jnp.float32,
        )
        + b_ref[...]
    )


def _fuse_table(table, W, b2d):
    vocab = table.shape[0]
    out_dim = W.shape[0]
    return pl.pallas_call(
        _fuse_table_body,
        out_shape=jax.ShapeDtypeStruct((vocab, out_dim), jnp.float32),
    )(table, W, b2d)


def _make_gather(n_idx, vocab):
    per_w = n_idx // _NUM_WORKERS
    assert n_idx % (_NUM_WORKERS * _CHUNK) == 0
    n_chunks = per_w // _CHUNK
    n_blocks = _CHUNK // _BLOCK_ROWS

    mesh = plsc.VectorSubcoreMesh(
        core_axis_name="c", subcore_axis_name="s",
        num_cores=_NUM_CORES, num_subcores=_NUM_SUBCORES,
    )

    @functools.partial(
        pl.kernel,
        mesh=mesh,
        compiler_params=pltpu.CompilerParams(
            use_tc_tiling_on_sc=False, needs_layout_passes=False),
        out_type=jax.ShapeDtypeStruct((n_idx * _D,), jnp.float32),
        scratch_types=[
            pltpu.VMEM((vocab * _D,), jnp.float32),
            pltpu.VMEM((_CHUNK,), jnp.int32),
            pltpu.VMEM((_CHUNK * _D,), jnp.float32),
            pltpu.VMEM((2 * _PHASES * _LANES,), jnp.int32),
        ],
    )
    def gather_kernel(idx_hbm, tab_hbm, pat_hbm, out_hbm,
                      tab_v, idx_v, out_v, pat_v):
        wid = lax.axis_index("s") * _NUM_CORES + lax.axis_index("c")
        base = wid * per_w
        pltpu.sync_copy(tab_hbm, tab_v)
        pltpu.sync_copy(pat_hbm, pat_v)

        def chunk_body(i, carry):
            off = base + i * _CHUNK
            pltpu.sync_copy(idx_hbm.at[pl.ds(off, _CHUNK)], idx_v)

            def block_body(blk, carry2):
                row0 = blk * _BLOCK_ROWS
                e0 = blk * (_BLOCK_ROWS * _D)
                for p in range(_PHASES):
                    rp = pat_v[pl.ds(p * _LANES, _LANES)]
                    cp = pat_v[pl.ds((_PHASES + p) * _LANES, _LANES)]
                    rows = plsc.load_gather(idx_v, [rp + row0])
                    src = rows * _D + cp
                    vals = plsc.load_gather(tab_v, [src])
                    out_v[pl.ds(e0 + p * _LANES, _LANES)] = vals
                return carry2

            lax.fori_loop(0, n_blocks, block_body, 0)
            pltpu.sync_copy(out_v, out_hbm.at[pl.ds(off * _D, _CHUNK * _D)])
            return carry

        lax.fori_loop(0, n_chunks, chunk_body, 0)

    return gather_kernel


def kernel(x, table, W, b):
    batch, hist = x.shape
    out_dim = W.shape[0]
    vocab = table.shape[0]
    fused = _fuse_table(table, W, b.reshape(1, -1))
    n_idx = batch * hist
    gather = _make_gather(n_idx, vocab)
    e = np.arange(_PHASES * _LANES, dtype=np.int32)
    pats = jnp.asarray(np.concatenate([e // _D, e % _D]))
    out = gather(x.reshape(-1), fused.reshape(-1), pats)
    return out.reshape(batch, hist, out_dim)


# speedup vs baseline: 37.6037x; 10.5911x over previous
import functools

import jax
import jax.numpy as jnp
from jax import lax
from jax.experimental import pallas as pl
from jax.experimental.pallas import tpu as pltpu
from jax.experimental.pallas import tpu_sc as plsc

_NUM_CORES = 2
_NUM_SUBCORES = 16
_NUM_WORKERS = _NUM_CORES * _NUM_SUBCORES
_LANES = 16

_D = 10
_TAB_ROWS = 16
_TAB_COLS = 1024
_BTILE = 128
_LGROUP = 40


def _fuse_table_body(table_ref, w_ref, b_ref, out_ref):
    fused_t = (
        lax.dot_general(
            w_ref[...], table_ref[...],
            dimension_numbers=(((1,), (1,)), ((), ())),
            preferred_element_type=jnp.float32,
        )
        + b_ref[...]
    )
    out_ref[...] = jnp.pad(
        fused_t,
        ((0, _TAB_ROWS - fused_t.shape[0]),
         (0, _TAB_COLS - fused_t.shape[1])))


def _fuse_table(table, W, b_col):
    return pl.pallas_call(
        _fuse_table_body,
        out_shape=jax.ShapeDtypeStruct((_TAB_ROWS, _TAB_COLS), jnp.float32),
    )(table, W, b_col)


def _make_gather(batch, hist):
    assert batch % (_NUM_WORKERS * _BTILE) == 0
    assert hist % _LGROUP == 0
    bcols_per_w = batch // _BTILE // _NUM_WORKERS
    lgroups = hist // _LGROUP

    mesh = plsc.VectorSubcoreMesh(
        core_axis_name="c", subcore_axis_name="s",
        num_cores=_NUM_CORES, num_subcores=_NUM_SUBCORES,
    )

    @functools.partial(
        pl.kernel,
        mesh=mesh,
        compiler_params=pltpu.CompilerParams(needs_layout_passes=False),
        out_type=jax.ShapeDtypeStruct((_D, hist, batch), jnp.float32),
        scratch_types=[
            pltpu.VMEM((_TAB_ROWS, _TAB_COLS), jnp.float32),
            pltpu.VMEM((_LGROUP, _BTILE), jnp.int32),
            pltpu.VMEM((_D, _LGROUP, _BTILE), jnp.float32),
        ],
    )
    def gather_kernel(idx_hbm, tab_hbm, out_hbm, tab_v, idx_v, out_v):
        wid = lax.axis_index("s") * _NUM_CORES + lax.axis_index("c")
        pltpu.sync_copy(tab_hbm, tab_v)

        def unit_body(u, carry):
            g = u // lgroups
            lg = u % lgroups
            b0 = (wid * bcols_per_w + g) * _BTILE
            l0 = lg * _LGROUP
            pltpu.sync_copy(
                idx_hbm.at[pl.ds(l0, _LGROUP), pl.ds(b0, _BTILE)], idx_v)

            def l_body(l, carry2):
                for bb in range(_BTILE // _LANES):
                    idx16 = idx_v[l, pl.ds(bb * _LANES, _LANES)]
                    for d in range(_D):
                        dv = jnp.full((_LANES,), d, jnp.int32)
                        vals = plsc.load_gather(tab_v, [dv, idx16])
                        out_v[d, l, pl.ds(bb * _LANES, _LANES)] = vals
                return carry2

            lax.fori_loop(0, _LGROUP, l_body, 0)
            pltpu.sync_copy(
                out_v,
                out_hbm.at[:, pl.ds(l0, _LGROUP), pl.ds(b0, _BTILE)])
            return carry

        lax.fori_loop(0, bcols_per_w * lgroups, unit_body, 0)

    return gather_kernel


def kernel(x, table, W, b):
    batch, hist = x.shape
    fused_t = _fuse_table(table, W, b.reshape(-1, 1))
    gather = _make_gather(batch, hist)
    out_t = gather(x.T, fused_t)
    return jnp.transpose(out_t, (2, 1, 0))


# speedup vs baseline: 42.0626x; 1.1186x over previous
import functools

import jax
import jax.numpy as jnp
from jax import lax
from jax.experimental import pallas as pl
from jax.experimental.pallas import tpu as pltpu
from jax.experimental.pallas import tpu_sc as plsc

_NUM_CORES = 2
_NUM_SUBCORES = 16
_NUM_WORKERS = _NUM_CORES * _NUM_SUBCORES
_LANES = 16

_D = 10
_TAB_ROWS = 16
_TAB_COLS = 1024
_UB = 256
_UL = 8


def _fuse_table_body(table_ref, w_ref, b_ref, out_ref):
    fused_t = (
        lax.dot_general(
            w_ref[...], table_ref[...],
            dimension_numbers=(((1,), (1,)), ((), ())),
            preferred_element_type=jnp.float32,
        )
        + b_ref[...]
    )
    out_ref[...] = jnp.pad(
        fused_t,
        ((0, _TAB_ROWS - fused_t.shape[0]),
         (0, _TAB_COLS - fused_t.shape[1])))


def _fuse_table(table, W, b_col):
    return pl.pallas_call(
        _fuse_table_body,
        out_shape=jax.ShapeDtypeStruct((_TAB_ROWS, _TAB_COLS), jnp.float32),
    )(table, W, b_col)


def _make_gather(batch, hist):
    bcol = batch // _NUM_WORKERS
    assert batch % (_NUM_WORKERS * _UB) == 0
    assert bcol % _UB == 0 and hist % _UL == 0
    halves = bcol // _UB
    lgroups = hist // _UL
    n_units = halves * lgroups
    assert n_units % 2 == 0

    mesh = plsc.VectorSubcoreMesh(
        core_axis_name="c", subcore_axis_name="s",
        num_cores=_NUM_CORES, num_subcores=_NUM_SUBCORES,
    )

    @functools.partial(
        pl.kernel,
        mesh=mesh,
        compiler_params=pltpu.CompilerParams(needs_layout_passes=False),
        out_type=jax.ShapeDtypeStruct((_D, hist, batch), jnp.float32),
        scratch_types=[
            pltpu.VMEM((_TAB_ROWS, _TAB_COLS), jnp.float32),
            pltpu.VMEM((_UL, _UB), jnp.int32),
            pltpu.VMEM((_UL, _UB), jnp.int32),
            pltpu.VMEM((_D, _UL, _UB), jnp.float32),
            pltpu.VMEM((_D, _UL, _UB), jnp.float32),
            pltpu.SemaphoreType.DMA,
            pltpu.SemaphoreType.DMA,
            pltpu.SemaphoreType.DMA,
            pltpu.SemaphoreType.DMA,
        ],
    )
    def gather_kernel(idx_hbm, tab_hbm, out_hbm,
                      tab_v, idx_a, idx_b, out_a, out_b,
                      sin_a, sin_b, sout_a, sout_b):
        wid = lax.axis_index("s") * _NUM_CORES + lax.axis_index("c")
        wb0 = wid * bcol
        pltpu.sync_copy(tab_hbm, tab_v)
        dvs = [jnp.full((_LANES,), d, jnp.int32) for d in range(_D)]

        def unit_slices(u):
            lg = u // halves
            half = u % halves
            b0 = wb0 + half * _UB
            l0 = lg * _UL
            return (idx_hbm.at[pl.ds(l0, _UL), pl.ds(b0, _UB)],
                    out_hbm.at[:, pl.ds(l0, _UL), pl.ds(b0, _UB)])

        i0, _ = unit_slices(0)
        pltpu.async_copy(i0, idx_a, sin_a)
        i1, _ = unit_slices(1)
        pltpu.async_copy(i1, idx_b, sin_b)

        def pair_body(g, carry):
            for j, idx_v, out_v, sin, sout in (
                    (0, idx_a, out_a, sin_a, sout_a),
                    (1, idx_b, out_b, sin_b, sout_b)):
                u = 2 * g + j
                isl, osl = unit_slices(u)
                pltpu.make_async_copy(isl, idx_v, sin).wait()
                @pl.when(g >= 1)
                def _():
                    _, osl_prev = unit_slices(u - 2)
                    pltpu.make_async_copy(out_v, osl_prev, sout).wait()

                def l_body(l, carry2):
                    for bb in range(_UB // _LANES):
                        idx16 = idx_v[l, pl.ds(bb * _LANES, _LANES)]
                        for d in range(_D):
                            vals = plsc.load_gather(tab_v, [dvs[d], idx16])
                            out_v[d, l, pl.ds(bb * _LANES, _LANES)] = vals
                    return carry2

                lax.fori_loop(0, _UL, l_body, 0)
                pltpu.async_copy(out_v, osl, sout)

                @pl.when(g <= n_units // 2 - 2)
                def _():
                    isl_next, _ = unit_slices(u + 2)
                    pltpu.async_copy(isl_next, idx_v, sin)
            return carry

        lax.fori_loop(0, n_units // 2, pair_body, 0)
        _, osl_a = unit_slices(n_units - 2)
        pltpu.make_async_copy(out_a, osl_a, sout_a).wait()
        _, osl_b = unit_slices(n_units - 1)
        pltpu.make_async_copy(out_b, osl_b, sout_b).wait()

    return gather_kernel


def kernel(x, table, W, b):
    batch, hist = x.shape
    fused_t = _fuse_table(table, W, b.reshape(-1, 1))
    gather = _make_gather(batch, hist)
    out_t = gather(x.T, fused_t)
    return jnp.transpose(out_t, (2, 1, 0))


# speedup vs baseline: 96.5830x; 2.2962x over previous
import functools

import jax
import jax.numpy as jnp
from jax import lax
from jax.experimental import pallas as pl
from jax.experimental.pallas import tpu as pltpu
from jax.experimental.pallas import tpu_sc as plsc

_NUM_CORES = 2
_NUM_SUBCORES = 16
_NUM_WORKERS = _NUM_CORES * _NUM_SUBCORES
_LANES = 16

_D = 10
_TAB_ROWS = 16
_TAB_COLS = 1024
_UB = 256
_UL = 8


def _fuse_table_body(table_ref, w_ref, b_ref, out_ref):
    fused_t = (
        lax.dot_general(
            w_ref[...], table_ref[...],
            dimension_numbers=(((1,), (1,)), ((), ())),
            preferred_element_type=jnp.float32,
        )
        + b_ref[...]
    )
    out_ref[...] = jnp.pad(
        fused_t,
        ((0, _TAB_ROWS - fused_t.shape[0]),
         (0, _TAB_COLS - fused_t.shape[1])))


def _fuse_table(table, W, b_col):
    return pl.pallas_call(
        _fuse_table_body,
        out_shape=jax.ShapeDtypeStruct((_TAB_ROWS, _TAB_COLS), jnp.float32),
    )(table, W, b_col)


def _make_gather(batch, hist):
    bcol = batch // _NUM_WORKERS
    assert batch % (_NUM_WORKERS * _UB) == 0
    assert bcol % _UB == 0 and hist % _UL == 0
    halves = bcol // _UB
    lgroups = hist // _UL
    n_units = halves * lgroups
    assert n_units % 2 == 0

    mesh = plsc.VectorSubcoreMesh(
        core_axis_name="c", subcore_axis_name="s",
        num_cores=_NUM_CORES, num_subcores=_NUM_SUBCORES,
    )

    @functools.partial(
        pl.kernel,
        mesh=mesh,
        compiler_params=pltpu.CompilerParams(needs_layout_passes=False),
        out_type=jax.ShapeDtypeStruct((_D, hist, batch), jnp.float32),
        scratch_types=[
            pltpu.VMEM((_TAB_ROWS, _TAB_COLS), jnp.float32),
            pltpu.VMEM((_UL, _UB), jnp.int32),
            pltpu.VMEM((_UL, _UB), jnp.int32),
            pltpu.VMEM((_D, _UL, _UB), jnp.float32),
            pltpu.VMEM((_D, _UL, _UB), jnp.float32),
            pltpu.SemaphoreType.DMA,
            pltpu.SemaphoreType.DMA,
            pltpu.SemaphoreType.DMA,
            pltpu.SemaphoreType.DMA,
        ],
    )
    def gather_kernel(idx_hbm, tab_hbm, out_hbm,
                      tab_v, idx_a, idx_b, out_a, out_b,
                      sin_a, sin_b, sout_a, sout_b):
        wid = lax.axis_index("s") * _NUM_CORES + lax.axis_index("c")
        wb0 = wid * bcol
        pltpu.sync_copy(tab_hbm, tab_v)
        dvs = [jnp.full((_LANES,), d, jnp.int32) for d in range(_D)]

        def unit_slices(u):
            lg = u // halves
            half = u % halves
            b0 = wb0 + half * _UB
            l0 = lg * _UL
            return (idx_hbm.at[pl.ds(l0, _UL), pl.ds(b0, _UB)],
                    out_hbm.at[:, pl.ds(l0, _UL), pl.ds(b0, _UB)])

        i0, _ = unit_slices(0)
        pltpu.async_copy(i0, idx_a, sin_a)
        i1, _ = unit_slices(1)
        pltpu.async_copy(i1, idx_b, sin_b)

        def pair_body(g, carry):
            for j, idx_v, out_v, sin, sout in (
                    (0, idx_a, out_a, sin_a, sout_a),
                    (1, idx_b, out_b, sin_b, sout_b)):
                u = 2 * g + j
                isl, osl = unit_slices(u)
                pltpu.make_async_copy(isl, idx_v, sin).wait()
                @pl.when(g >= 1)
                def _():
                    _, osl_prev = unit_slices(u - 2)
                    pltpu.make_async_copy(out_v, osl_prev, sout).wait()

                @plsc.parallel_loop(0, _UL * (_UB // _LANES), 1, unroll=2)
                def l_body(i):
                    l = i >> 4
                    boff = (i & 15) * _LANES
                    idx16 = idx_v[l, pl.ds(boff, _LANES)]
                    vals = [plsc.load_gather(tab_v, [dvs[d], idx16])
                            for d in range(_D)]
                    for d in range(_D):
                        out_v[d, l, pl.ds(boff, _LANES)] = vals[d]
                pltpu.async_copy(out_v, osl, sout)

                @pl.when(g <= n_units // 2 - 2)
                def _():
                    isl_next, _ = unit_slices(u + 2)
                    pltpu.async_copy(isl_next, idx_v, sin)
            return carry

        lax.fori_loop(0, n_units // 2, pair_body, 0)
        _, osl_a = unit_slices(n_units - 2)
        pltpu.make_async_copy(out_a, osl_a, sout_a).wait()
        _, osl_b = unit_slices(n_units - 1)
        pltpu.make_async_copy(out_b, osl_b, sout_b).wait()

    return gather_kernel


def kernel(x, table, W, b):
    batch, hist = x.shape
    fused_t = _fuse_table(table, W, b.reshape(-1, 1))
    gather = _make_gather(batch, hist)
    out_t = gather(x.T, fused_t)
    return jnp.transpose(out_t, (2, 1, 0))
